# baseline (speedup 1.0000x reference)
import jax
import jax.numpy as jnp
from jax import lax
from jax.experimental import pallas as pl
from jax.experimental.pallas import tpu as pltpu

S, IN_F, OUT_F, E = 2048, 768, 768, 8


def _fused_body(x_ref, rw_ref, w_ref, scale_ref, masks_ref, o_ref,
                xw_s, idx_s, wr_s):
    e = pl.program_id(0)

    @pl.when(e == 0)
    def _():
        x = x_ref[...]
        logits = lax.dot_general(x, rw_ref[...], (((1,), (1,)), ((), ())),
                                 preferred_element_type=jnp.float32)
        col = lax.broadcasted_iota(jnp.int32, (S, E), 1)
        m = jnp.max(logits, axis=1, keepdims=True)
        top1w = 1.0 / jnp.sum(jnp.exp(logits - m), axis=1, keepdims=True)
        idx_s[...] = jnp.min(jnp.where(logits >= m, col, E), axis=1,
                             keepdims=True)
        xw_s[...] = (x * top1w).astype(jnp.bfloat16)
        wr_s[...] = (jnp.clip(jnp.round(w_ref[...] * 2.0), -1.0, 1.0)
                     * scale_ref[...]).astype(jnp.bfloat16)

    wm = wr_s[...] * masks_ref[0].astype(jnp.bfloat16)
    ye = lax.dot_general(xw_s[...], wm, (((1,), (1,)), ((), ())),
                         preferred_element_type=jnp.float32)
    yv = jnp.where(idx_s[...] == e, ye, 0.0)

    @pl.when(e == 0)
    def _():
        o_ref[...] = yv

    @pl.when(e > 0)
    def _():
        o_ref[...] += yv


def _fused_call(x2, router_w, weight, scale_col, masks, interpret=False):
    return pl.pallas_call(
        _fused_body,
        grid=(E,),
        in_specs=[
            pl.BlockSpec((S, IN_F), lambda e: (0, 0)),
            pl.BlockSpec((E, IN_F), lambda e: (0, 0)),
            pl.BlockSpec((OUT_F, IN_F), lambda e: (0, 0)),
            pl.BlockSpec((OUT_F, 1), lambda e: (0, 0)),
            pl.BlockSpec((1, OUT_F, IN_F), lambda e: (e, 0, 0)),
        ],
        out_specs=pl.BlockSpec((S, OUT_F), lambda e: (0, 0)),
        out_shape=jax.ShapeDtypeStruct((S, OUT_F), jnp.float32),
        scratch_shapes=[
            pltpu.VMEM((S, IN_F), jnp.bfloat16),
            pltpu.VMEM((S, 1), jnp.int32),
            pltpu.VMEM((OUT_F, IN_F), jnp.bfloat16),
        ],
        interpret=interpret,
    )(x2, router_w, weight, scale_col, masks)


@jax.jit
def kernel(x, weight, scale, threshold, expert_masks, router_w):
    del threshold
    out = _fused_call(x.reshape(S, IN_F), router_w, weight,
                      scale.reshape(OUT_F, 1), expert_masks)
    return out.reshape(1, S, OUT_F)


# speedup vs baseline: 1.1267x; 1.1267x over previous
import jax
import jax.numpy as jnp
from jax import lax
from jax.experimental import pallas as pl
from jax.experimental.pallas import tpu as pltpu

S, IN_F, OUT_F, E = 2048, 768, 768, 8


def _fused_body(x_ref, rw_ref, w_ref, scale_ref, masks_ref, o_ref,
                xw_s, idx_s, wr_s):
    e = pl.program_id(0)

    @pl.when(e == 0)
    def _():
        x = x_ref[...]
        logits = lax.dot_general(x, rw_ref[...], (((1,), (1,)), ((), ())),
                                 preferred_element_type=jnp.float32)
        col = lax.broadcasted_iota(jnp.int32, (S, E), 1)
        m = jnp.max(logits, axis=1, keepdims=True)
        top1w = 1.0 / jnp.sum(jnp.exp(logits - m), axis=1, keepdims=True)
        idx_s[...] = jnp.min(jnp.where(logits >= m, col, E), axis=1,
                             keepdims=True)
        xw_s[...] = (x * top1w).astype(jnp.bfloat16)
        wr_s[...] = (jnp.clip(jnp.round(w_ref[...] * 2.0), -1.0, 1.0)
                     * scale_ref[...]).astype(jnp.bfloat16)
        o_ref[...] = jnp.zeros((S, OUT_F), jnp.float32)

    wm = wr_s[...] * masks_ref[0].astype(jnp.bfloat16)
    ye = lax.dot_general(xw_s[...], wm, (((1,), (1,)), ((), ())),
                         preferred_element_type=jnp.float32)
    sel = (idx_s[...] == e).astype(jnp.float32)
    o_ref[...] += sel * ye


def _fused_call(x2, router_w, weight, scale_col, masks, interpret=False):
    return pl.pallas_call(
        _fused_body,
        grid=(E,),
        in_specs=[
            pl.BlockSpec((S, IN_F), lambda e: (0, 0)),
            pl.BlockSpec((E, IN_F), lambda e: (0, 0)),
            pl.BlockSpec((OUT_F, IN_F), lambda e: (0, 0)),
            pl.BlockSpec((OUT_F, 1), lambda e: (0, 0)),
            pl.BlockSpec((1, OUT_F, IN_F), lambda e: (e, 0, 0)),
        ],
        out_specs=pl.BlockSpec((S, OUT_F), lambda e: (0, 0)),
        out_shape=jax.ShapeDtypeStruct((S, OUT_F), jnp.float32),
        scratch_shapes=[
            pltpu.VMEM((S, IN_F), jnp.bfloat16),
            pltpu.VMEM((S, 1), jnp.int32),
            pltpu.VMEM((OUT_F, IN_F), jnp.bfloat16),
        ],
        interpret=interpret,
    )(x2, router_w, weight, scale_col, masks)


@jax.jit
def kernel(x, weight, scale, threshold, expert_masks, router_w):
    del threshold
    out = _fused_call(x.reshape(S, IN_F), router_w, weight,
                      scale.reshape(OUT_F, 1), expert_masks)
    return out.reshape(1, S, OUT_F)
